# baseline (speedup 1.0000x reference)
import jax
import jax.numpy as jnp
from jax.experimental import pallas as pl

N = 50000
G = 512
L = 12


def _gcn_conv(x, src, dst, ew, W, b, num_nodes):
    x = x @ W
    deg = jax.ops.segment_sum(ew, dst, num_segments=num_nodes)
    dis = jnp.where(deg > 0, jax.lax.rsqrt(jnp.maximum(deg, 1e-12)), 0.0)
    norm = dis[src] * ew * dis[dst]
    out = jax.ops.segment_sum(x[src] * norm[:, None], dst, num_segments=num_nodes)
    return out + b


def _div_kernel(s_ref, c_ref, o_ref):
    o_ref[...] = s_ref[...] / jnp.maximum(c_ref[...], 1.0)


def kernel(partial_graph_node_categorical_features, node_features, edge_index,
           edge_features, graph_to_focus_node_map, candidate_attachment_points,
           batch_index, embed_table, W_first, b_first, Ws, bs):
    motif = jnp.take(embed_table, partial_graph_node_categorical_features, axis=0)
    h0 = jnp.concatenate([node_features, motif], axis=-1)
    focus = jnp.concatenate([graph_to_focus_node_map, candidate_attachment_points], axis=0)
    bit = jnp.zeros((node_features.shape[0], 1), jnp.float32).at[focus].add(
        jnp.ones((focus.shape[0], 1), jnp.float32))
    bit = jnp.minimum(bit, jnp.ones((1,), jnp.float32))
    h = jnp.concatenate([h0, bit], axis=-1)
    loop = jnp.arange(N)
    src = jnp.concatenate([edge_index[0], loop])
    dst = jnp.concatenate([edge_index[1], loop])
    ew = jnp.concatenate([edge_features.astype(jnp.float32), jnp.ones((N,), jnp.float32)])
    x = jax.nn.relu(h @ W_first + b_first)
    reps = [x]
    for i in range(L):
        x = jax.nn.relu(_gcn_conv(x, src, dst, ew, Ws[i], bs[i], N))
        reps.append(x)
    node_representations = jnp.concatenate(reps, axis=-1)
    sums = jax.ops.segment_sum(node_representations, batch_index, num_segments=G)
    counts = jax.ops.segment_sum(jnp.ones((N,), jnp.float32), batch_index, num_segments=G)
    graph_representations = pl.pallas_call(
        _div_kernel,
        out_shape=jax.ShapeDtypeStruct(sums.shape, sums.dtype),
    )(sums, jnp.broadcast_to(counts[:, None], sums.shape))
    return (graph_representations, node_representations)


# speedup vs baseline: 4.6740x; 4.6740x over previous
import functools

import jax
import jax.numpy as jnp
from jax import lax
from jax.experimental import pallas as pl
from jax.experimental.pallas import tpu as pltpu
from jax.experimental.pallas import tpu_sc as plsc

N = 50000
E = 800000
F_IN = 32
EMB = 64
H = 64
L = 12
VOCAB = 139
G = 512
NCAND = 2048

E_PAD = 819200
ER = E_PAD // 128
ROWS_PER_TILE = ER // 16
ROWS_PER_WORKER = ER // 32
NPT = 3128
GPAD = 528
BR = 416
WROWS = 13

_mesh = plsc.VectorSubcoreMesh(core_axis_name="c", subcore_axis_name="s")


def _zero16():
    return jnp.zeros((16,), jnp.float32)


@functools.partial(
    pl.kernel,
    out_type=(jax.ShapeDtypeStruct((32, 1, N), jnp.float32),
              jax.ShapeDtypeStruct((32, 1, N), jnp.float32)),
    mesh=_mesh,
    compiler_params=pltpu.CompilerParams(needs_layout_passes=False, use_tc_tiling_on_sc=False),
    scratch_types=[
        pltpu.VMEM((N,), jnp.float32),
        pltpu.VMEM((N,), jnp.float32),
        pltpu.VMEM((40, 128), jnp.int32),
        pltpu.VMEM((40, 128), jnp.float32),
        pltpu.VMEM((1, 128), jnp.int32),
    ],
)
def _stats_kernel(dst2d, ew2d, focus3d, deg_out, cnt_out,
                  acc_d, acc_f, dbuf, wbuf, fbuf):
    ci = lax.axis_index("c")
    si = lax.axis_index("s")
    w = ci * 16 + si

    z16i = jnp.zeros((16,), jnp.int32)

    def zero_body(i, _):
        acc_d[pl.ds(16 * i, 16)] = _zero16()
        acc_f[pl.ds(16 * i, 16)] = _zero16()
        return _
    lax.fori_loop(0, N // 16, zero_body, None)

    base = w * ROWS_PER_WORKER

    def chunk_body(c, _):
        pltpu.sync_copy(dst2d.at[pl.ds(base + 40 * c, 40)], dbuf)
        pltpu.sync_copy(ew2d.at[pl.ds(base + 40 * c, 40)], wbuf)

        def row_body(j, _):
            for g in range(8):
                d16 = dbuf[j, pl.ds(16 * g, 16)]
                w16 = wbuf[j, pl.ds(16 * g, 16)]
                plsc.addupdate_scatter(acc_d, [d16], w16)
            return _
        lax.fori_loop(0, 40, row_body, None)
        return _
    lax.fori_loop(0, ROWS_PER_WORKER // 40, chunk_body, None)

    @pl.when(w < 20)
    def _():
        pltpu.sync_copy(focus3d.at[w], fbuf)
        ones = jnp.ones((16,), jnp.float32)
        for g in range(8):
            f16 = fbuf[0, pl.ds(16 * g, 16)]
            plsc.addupdate_scatter(acc_f, [f16], ones)

    pltpu.sync_copy(acc_d, deg_out.at[w].at[0])
    pltpu.sync_copy(acc_f, cnt_out.at[w].at[0])


@functools.partial(
    pl.kernel,
    out_type=jax.ShapeDtypeStruct((2 * N, H // 2), jnp.float32),
    mesh=_mesh,
    compiler_params=pltpu.CompilerParams(needs_layout_passes=False, use_tc_tiling_on_sc=False),
    scratch_types=[
        pltpu.VMEM((8, 128), jnp.int32),
        pltpu.VMEM((8, 128), jnp.int32),
        pltpu.VMEM((8, 128), jnp.float32),
        pltpu.VMEM((128, H // 2), jnp.float32),
        pltpu.VMEM_SHARED((N, H // 2), jnp.float32),
    ],
)
def _agg_kernel(y_cat, src2d, dst2d, ew2d, agg_out,
                sbuf, dbuf, wbuf, gbuf, acc_s):
    ci = lax.axis_index("c")
    si = lax.axis_index("s")
    coff = ci * N

    nbase = pl.multiple_of(jnp.where(si == 15, N - NPT, NPT * si), 8)
    pltpu.sync_copy(y_cat.at[pl.ds(coff + nbase, NPT)],
                    acc_s.at[pl.ds(nbase, NPT)])
    plsc.subcore_barrier()

    ebase = si * ROWS_PER_TILE

    def chunk_body(c, _):
        r0 = ebase + 8 * c
        pltpu.sync_copy(src2d.at[pl.ds(r0, 8)], sbuf)
        pltpu.sync_copy(dst2d.at[pl.ds(r0, 8)], dbuf)
        pltpu.sync_copy(ew2d.at[pl.ds(r0, 8)], wbuf)
        off16 = jnp.full((16,), coff, jnp.int32)
        for jr in range(8):
            for g in range(8):
                sl = pl.ds(16 * g, 16)
                sbuf[jr, sl] = sbuf[jr, sl] + off16
        for j in range(8):
            pltpu.sync_copy(y_cat.at[sbuf.at[j]], gbuf)

            def scale_body(e, _):
                w16 = plsc.load_gather(
                    wbuf, [jnp.full((16,), j, jnp.int32),
                           jnp.full((16,), e, jnp.int32)])
                gbuf[e, pl.ds(0, 16)] = gbuf[e, pl.ds(0, 16)] * w16
                gbuf[e, pl.ds(16, 16)] = gbuf[e, pl.ds(16, 16)] * w16
                return _
            lax.fori_loop(0, 128, scale_body, None)
            pltpu.sync_copy(gbuf, acc_s.at[dbuf.at[j]], add=True)
        return _
    lax.fori_loop(0, ROWS_PER_TILE // 8, chunk_body, None)

    plsc.subcore_barrier()
    pltpu.sync_copy(acc_s.at[pl.ds(nbase, NPT)],
                    agg_out.at[pl.ds(coff + nbase, NPT)])


@functools.partial(
    pl.kernel,
    out_type=(jax.ShapeDtypeStruct((26, GPAD, H), jnp.float32),
              jax.ShapeDtypeStruct((32, 1, GPAD), jnp.float32)),
    mesh=_mesh,
    compiler_params=pltpu.CompilerParams(needs_layout_passes=False, use_tc_tiling_on_sc=False),
    scratch_types=[
        pltpu.VMEM((128, H), jnp.float32),
        pltpu.VMEM((1, 128), jnp.int32),
        pltpu.VMEM((GPAD,), jnp.float32),
        pltpu.VMEM_SHARED((13, GPAD, H), jnp.float32),
    ],
)
def _readout_kernel(*args):
    xs = args[:13]
    batch3d, z528 = args[13], args[14]
    sums_out, cnt_out = args[15], args[16]
    xbuf, ibuf, cntv, acc_r = args[17:]

    ci = lax.axis_index("c")
    si = lax.axis_index("s")
    w = ci * 16 + si

    @pl.when(si == 0)
    def _():
        for i in range(13):
            pltpu.sync_copy(z528, acc_r.at[i])

    z16i = jnp.zeros((16,), jnp.int32)

    def zero_body(i, _):
        cntv[pl.ds(16 * i, 16)] = _zero16()
        return _
    lax.fori_loop(0, GPAD // 16, zero_body, None)
    plsc.subcore_barrier()

    ones = jnp.ones((16,), jnp.float32)
    for b in range(WROWS):
        row = w * WROWS + b
        nb = pl.multiple_of(row * 128, 8)

        @pl.when(nb < N)
        def _(row=row, nb=nb):
            pltpu.sync_copy(batch3d.at[row], ibuf)

            @pl.when(nb + 128 <= N)
            def _():
                for i in range(13):
                    pltpu.sync_copy(xs[i].at[pl.ds(nb, 128)], xbuf)
                    pltpu.sync_copy(xbuf, acc_r.at[i].at[ibuf.at[0]], add=True)

            @pl.when(nb + 128 > N)
            def _():
                for i in range(13):
                    pltpu.sync_copy(xs[i].at[pl.ds(nb, N % 128)],
                                    xbuf.at[pl.ds(0, N % 128)])
                    pltpu.sync_copy(xbuf, acc_r.at[i].at[ibuf.at[0]], add=True)

            for g in range(8):
                b16 = ibuf[0, pl.ds(16 * g, 16)]
                plsc.addupdate_scatter(cntv, [b16], ones)

    plsc.subcore_barrier()
    pltpu.sync_copy(cntv, cnt_out.at[w].at[0])

    @pl.when(si < 13)
    def _():
        pltpu.sync_copy(acc_r.at[si], sums_out.at[ci * 13 + si])


def _reduce32_body(dp_ref, cp_ref, do_ref, co_ref):
    @pl.when(pl.program_id(1) == 0)
    def _():
        do_ref[...] = jnp.zeros_like(do_ref)
        co_ref[...] = jnp.zeros_like(co_ref)
    do_ref[...] += dp_ref[0]
    co_ref[...] += cp_ref[0]


def _first_body(nf_ref, cat_ref, degp_ref, cntp_ref, emb_ref, wf_ref, bf_ref,
                w1_ref, x0_ref, dis_ref, y1_ref):
    deg = degp_ref[...] + 1.0
    dis = lax.rsqrt(deg)
    bit = jnp.minimum(cntp_ref[...], 1.0)
    cat = cat_ref[...]
    iota = lax.broadcasted_iota(jnp.int32, (1, VOCAB), 1)
    onehot = (cat == iota).astype(jnp.float32)
    tbl = jnp.dot(emb_ref[...], wf_ref[pl.ds(F_IN, EMB), :],
                  preferred_element_type=jnp.float32)
    x = jnp.dot(nf_ref[...], wf_ref[pl.ds(0, F_IN), :],
                preferred_element_type=jnp.float32)
    x = x + jnp.dot(onehot, tbl, preferred_element_type=jnp.float32)
    x = x + bit * wf_ref[pl.ds(F_IN + EMB, 1), :] + bf_ref[...]
    x0 = jnp.maximum(x, 0.0)
    y = jnp.dot(x0, w1_ref[...], preferred_element_type=jnp.float32) * dis
    x0_ref[...] = x0
    dis_ref[...] = dis
    y1_ref[0] = y[:, :H // 2]
    y1_ref[1] = y[:, H // 2:]


def _layer_body(agg_ref, dis_ref, b_ref, wn_ref, x_ref, yn_ref):
    dis = dis_ref[...]
    a = jnp.concatenate([agg_ref[0], agg_ref[1]], axis=-1)
    x = jnp.maximum(a * dis + b_ref[...], 0.0)
    x_ref[...] = x
    yn = jnp.dot(x, wn_ref[...], preferred_element_type=jnp.float32) * dis
    yn_ref[0] = yn[:, :H // 2]
    yn_ref[1] = yn[:, H // 2:]


def _last_body(agg_ref, dis_ref, b_ref, x_ref):
    a = jnp.concatenate([agg_ref[0], agg_ref[1]], axis=-1)
    x_ref[...] = jnp.maximum(a * dis_ref[...] + b_ref[...], 0.0)


def _div_body(sums_ref, cnt_ref, out_ref):
    counts = jnp.sum(cnt_ref[...], axis=0)[:G]
    cmax = jnp.maximum(counts, 1.0)[:, None]
    for i in range(13):
        si = sums_ref[i, pl.ds(0, G), :] + sums_ref[13 + i, pl.ds(0, G), :]
        out_ref[:, pl.ds(H * i, H)] = si / cmax


_BN = 1000
_NG = N // _BN


def _full(shape):
    return pl.BlockSpec(shape, lambda i: tuple(0 for _ in shape))


def kernel(partial_graph_node_categorical_features, node_features, edge_index,
           edge_features, graph_to_focus_node_map, candidate_attachment_points,
           batch_index, embed_table, W_first, b_first, Ws, bs):
    f32 = jnp.float32
    src = edge_index[0].astype(jnp.int32)
    dst = edge_index[1].astype(jnp.int32)
    ew = edge_features.astype(f32)
    pad = E_PAD - E
    src2d = jnp.pad(src, (0, pad)).reshape(ER, 128)
    dst2d = jnp.pad(dst, (0, pad)).reshape(ER, 128)
    ew2d = jnp.pad(ew, (0, pad)).reshape(ER, 128)
    focus = jnp.concatenate([graph_to_focus_node_map,
                             candidate_attachment_points]).astype(jnp.int32)
    focus3d = focus.reshape(20, 1, 128)
    batch3d = jnp.pad(batch_index.astype(jnp.int32), (0, BR * 128 - N),
                      constant_values=G).reshape(BR, 1, 128)
    z528 = jnp.zeros((GPAD, H), f32)

    deg_p, cnt_p = _stats_kernel(dst2d, ew2d, focus3d)
    deg_p = deg_p.reshape(32, N, 1)
    cnt_p = cnt_p.reshape(32, N, 1)
    deg_r, cnt_r = pl.pallas_call(
        _reduce32_body,
        grid=(_NG, 32),
        in_specs=[
            pl.BlockSpec((1, _BN, 1), lambda i, k: (k, i, 0)),
            pl.BlockSpec((1, _BN, 1), lambda i, k: (k, i, 0)),
        ],
        out_specs=[
            pl.BlockSpec((_BN, 1), lambda i, k: (i, 0)),
            pl.BlockSpec((_BN, 1), lambda i, k: (i, 0)),
        ],
        out_shape=[
            jax.ShapeDtypeStruct((N, 1), f32),
            jax.ShapeDtypeStruct((N, 1), f32),
        ],
    )(deg_p, cnt_p)

    cat2 = partial_graph_node_categorical_features.astype(jnp.int32)[:, None]
    nf = node_features.astype(f32)

    x0, dis, y1 = pl.pallas_call(
        _first_body,
        grid=(_NG,),
        in_specs=[
            pl.BlockSpec((_BN, F_IN), lambda i: (i, 0)),
            pl.BlockSpec((_BN, 1), lambda i: (i, 0)),
            pl.BlockSpec((_BN, 1), lambda i: (i, 0)),
            pl.BlockSpec((_BN, 1), lambda i: (i, 0)),
            _full((VOCAB, EMB)),
            _full((F_IN + EMB + 1, H)),
            _full((1, H)),
            _full((H, H)),
        ],
        out_specs=[
            pl.BlockSpec((_BN, H), lambda i: (i, 0)),
            pl.BlockSpec((_BN, 1), lambda i: (i, 0)),
            pl.BlockSpec((2, _BN, H // 2), lambda i: (0, i, 0)),
        ],
        out_shape=[
            jax.ShapeDtypeStruct((N, H), f32),
            jax.ShapeDtypeStruct((N, 1), f32),
            jax.ShapeDtypeStruct((2, N, H // 2), f32),
        ],
    )(nf, cat2, deg_r, cnt_r, embed_table.astype(f32),
      W_first.astype(f32), b_first.astype(f32)[None, :], Ws[0].astype(f32))

    xs = [x0]
    y = y1
    for i in range(1, L + 1):
        agg = _agg_kernel(y.reshape(2 * N, H // 2), src2d, dst2d, ew2d)
        agg = agg.reshape(2, N, H // 2)
        bias = bs[i - 1].astype(f32)[None, :]
        if i < L:
            xi, y = pl.pallas_call(
                _layer_body,
                grid=(_NG,),
                in_specs=[
                    pl.BlockSpec((2, _BN, H // 2), lambda i: (0, i, 0)),
                    pl.BlockSpec((_BN, 1), lambda i: (i, 0)),
                    _full((1, H)),
                    _full((H, H)),
                ],
                out_specs=[
                    pl.BlockSpec((_BN, H), lambda i: (i, 0)),
                    pl.BlockSpec((2, _BN, H // 2), lambda i: (0, i, 0)),
                ],
                out_shape=[
                    jax.ShapeDtypeStruct((N, H), f32),
                    jax.ShapeDtypeStruct((2, N, H // 2), f32),
                ],
            )(agg, dis, bias, Ws[i].astype(f32))
        else:
            xi = pl.pallas_call(
                _last_body,
                grid=(_NG,),
                in_specs=[
                    pl.BlockSpec((2, _BN, H // 2), lambda i: (0, i, 0)),
                    pl.BlockSpec((_BN, 1), lambda i: (i, 0)),
                    _full((1, H)),
                ],
                out_specs=pl.BlockSpec((_BN, H), lambda i: (i, 0)),
                out_shape=jax.ShapeDtypeStruct((N, H), f32),
            )(agg, dis, bias)
        xs.append(xi)

    sums_p, cnt_parts = _readout_kernel(*xs, batch3d, z528)
    cnt_parts = cnt_parts.reshape(32, GPAD)

    graph_representations = pl.pallas_call(
        _div_body,
        grid=(1,),
        in_specs=[_full((26, GPAD, H)), _full((32, GPAD))],
        out_specs=_full((G, 13 * H)),
        out_shape=jax.ShapeDtypeStruct((G, 13 * H), f32),
    )(sums_p, cnt_parts)

    node_representations = jnp.concatenate(xs, axis=-1)
    return (graph_representations, node_representations)


# speedup vs baseline: 6.6383x; 1.4202x over previous
import functools

import jax
import jax.numpy as jnp
from jax import lax
from jax.experimental import pallas as pl
from jax.experimental.pallas import tpu as pltpu
from jax.experimental.pallas import tpu_sc as plsc

N = 50000
E = 800000
F_IN = 32
EMB = 64
H = 64
L = 12
VOCAB = 139
G = 512
NCAND = 2048

E_PAD = 819200
ER = E_PAD // 128
ROWS_PER_TILE = ER // 16
ROWS_PER_WORKER = ER // 32
NPT = 3128
GPAD = 528
BR = 416
WROWS = 13

_mesh = plsc.VectorSubcoreMesh(core_axis_name="c", subcore_axis_name="s")


def _zero16():
    return jnp.zeros((16,), jnp.float32)


@functools.partial(
    pl.kernel,
    out_type=(jax.ShapeDtypeStruct((32, 1, N), jnp.float32),
              jax.ShapeDtypeStruct((32, 1, N), jnp.float32)),
    mesh=_mesh,
    compiler_params=pltpu.CompilerParams(needs_layout_passes=False, use_tc_tiling_on_sc=False),
    scratch_types=[
        pltpu.VMEM((N,), jnp.float32),
        pltpu.VMEM((N,), jnp.float32),
        pltpu.VMEM((40, 128), jnp.int32),
        pltpu.VMEM((40, 128), jnp.float32),
        pltpu.VMEM((1, 128), jnp.int32),
    ],
)
def _stats_kernel(dst2d, ew2d, focus3d, deg_out, cnt_out,
                  acc_d, acc_f, dbuf, wbuf, fbuf):
    ci = lax.axis_index("c")
    si = lax.axis_index("s")
    w = ci * 16 + si

    z16i = jnp.zeros((16,), jnp.int32)

    def zero_body(i, _):
        acc_d[pl.ds(16 * i, 16)] = _zero16()
        acc_f[pl.ds(16 * i, 16)] = _zero16()
        return _
    lax.fori_loop(0, N // 16, zero_body, None)

    base = w * ROWS_PER_WORKER

    def chunk_body(c, _):
        pltpu.sync_copy(dst2d.at[pl.ds(base + 40 * c, 40)], dbuf)
        pltpu.sync_copy(ew2d.at[pl.ds(base + 40 * c, 40)], wbuf)

        def row_body(j, _):
            for g in range(8):
                d16 = dbuf[j, pl.ds(16 * g, 16)]
                w16 = wbuf[j, pl.ds(16 * g, 16)]
                plsc.addupdate_scatter(acc_d, [d16], w16)
            return _
        lax.fori_loop(0, 40, row_body, None)
        return _
    lax.fori_loop(0, ROWS_PER_WORKER // 40, chunk_body, None)

    @pl.when(w < 20)
    def _():
        pltpu.sync_copy(focus3d.at[w], fbuf)
        ones = jnp.ones((16,), jnp.float32)
        for g in range(8):
            f16 = fbuf[0, pl.ds(16 * g, 16)]
            plsc.addupdate_scatter(acc_f, [f16], ones)

    pltpu.sync_copy(acc_d, deg_out.at[w].at[0])
    pltpu.sync_copy(acc_f, cnt_out.at[w].at[0])


@functools.partial(
    pl.kernel,
    out_type=jax.ShapeDtypeStruct((2 * N, H // 2), jnp.float32),
    mesh=_mesh,
    compiler_params=pltpu.CompilerParams(needs_layout_passes=False, use_tc_tiling_on_sc=False),
    scratch_types=[
        pltpu.VMEM((2, 8, 128), jnp.int32),
        pltpu.VMEM((2, 8, 128), jnp.int32),
        pltpu.VMEM((2, 8, 128), jnp.float32),
        pltpu.VMEM((4, 128, H // 2), jnp.float32),
        pltpu.VMEM_SHARED((N, H // 2), jnp.float32),
        pltpu.SemaphoreType.DMA((4,)),
        pltpu.SemaphoreType.DMA((4,)),
    ],
)
def _agg_kernel(y_cat, src2d, dst2d, ew2d, agg_out,
                sbuf, dbuf, wbuf, gbuf, acc_s, sem_g, sem_s):
    ci = lax.axis_index("c")
    si = lax.axis_index("s")
    coff = ci * N

    nbase = pl.multiple_of(jnp.where(si == 15, N - NPT, NPT * si), 8)
    pltpu.sync_copy(y_cat.at[pl.ds(coff + nbase, NPT)],
                    acc_s.at[pl.ds(nbase, NPT)])
    plsc.subcore_barrier()

    ebase = si * ROWS_PER_TILE
    off16 = jnp.full((16,), coff, jnp.int32)
    RPT = ROWS_PER_TILE

    def idx_load(r0, q):
        pltpu.sync_copy(src2d.at[pl.ds(r0, 8)], sbuf.at[q])
        pltpu.sync_copy(dst2d.at[pl.ds(r0, 8)], dbuf.at[q])
        pltpu.sync_copy(ew2d.at[pl.ds(r0, 8)], wbuf.at[q])
        for jr in range(8):
            for g in range(8):
                sl = pl.ds(16 * g, 16)
                sbuf[q, jr, sl] = sbuf[q, jr, sl] + off16

    def g_issue(q, jr, slot):
        pltpu.async_copy(y_cat.at[sbuf.at[q].at[jr]], gbuf.at[slot],
                         sem_g.at[slot])

    def g_wait(slot):
        pltpu.make_async_copy(y_cat.at[sbuf.at[0].at[0]], gbuf.at[slot],
                              sem_g.at[slot]).wait()

    def s_issue(q, jr, slot):
        pltpu.async_copy(gbuf.at[slot], acc_s.at[dbuf.at[q].at[jr]],
                         sem_s.at[slot], add=True)

    def s_drain(slot):
        pltpu.make_async_copy(gbuf.at[slot], acc_s.at[dbuf.at[0].at[0]],
                              sem_s.at[slot]).wait()

    def scale(q, jr, slot):
        def scale_body(i, _):
            for u4 in range(4):
                e = 4 * i + u4
                w16 = plsc.load_gather(
                    wbuf.at[q], [jnp.full((16,), jr, jnp.int32),
                                 jnp.full((16,), e, jnp.int32)])
                gbuf[slot, e, pl.ds(0, 16)] = \
                    gbuf[slot, e, pl.ds(0, 16)] * w16
                gbuf[slot, e, pl.ds(16, 16)] = \
                    gbuf[slot, e, pl.ds(16, 16)] * w16
            return _
        lax.fori_loop(0, 32, scale_body, None)

    idx_load(ebase, 0)
    g_issue(0, 0, 0)

    def block_body(it, _):
        for u in range(16):
            m = 16 * it + u
            slot = u % 4
            nslot = (u + 1) % 4
            q = (u // 8) % 2
            qn = ((u + 1) // 8) % 2
            jr = u % 8

            @pl.when(m >= 3)
            def _():
                s_drain(nslot)
            if u in (7, 15):
                @pl.when(m + 8 < RPT)
                def _(m=m, qn=qn):
                    idx_load(pl.multiple_of(ebase + m + 1, 8), qn)

            @pl.when(m + 1 < RPT)
            def _():
                g_issue(qn, (u + 1) % 8, nslot)
            g_wait(slot)
            scale(q, jr, slot)
            s_issue(q, jr, slot)
        return _
    lax.fori_loop(0, RPT // 16, block_body, None)
    for slot in (1, 2, 3):
        s_drain(slot)

    plsc.subcore_barrier()
    pltpu.sync_copy(acc_s.at[pl.ds(nbase, NPT)],
                    agg_out.at[pl.ds(coff + nbase, NPT)])


@functools.partial(
    pl.kernel,
    out_type=(jax.ShapeDtypeStruct((26, GPAD, H), jnp.float32),
              jax.ShapeDtypeStruct((32, 1, GPAD), jnp.float32)),
    mesh=_mesh,
    compiler_params=pltpu.CompilerParams(needs_layout_passes=False, use_tc_tiling_on_sc=False),
    scratch_types=[
        pltpu.VMEM((128, H), jnp.float32),
        pltpu.VMEM((1, 128), jnp.int32),
        pltpu.VMEM((GPAD,), jnp.float32),
        pltpu.VMEM_SHARED((13, GPAD, H), jnp.float32),
    ],
)
def _readout_kernel(*args):
    xs = args[:13]
    batch3d, z528 = args[13], args[14]
    sums_out, cnt_out = args[15], args[16]
    xbuf, ibuf, cntv, acc_r = args[17:]

    ci = lax.axis_index("c")
    si = lax.axis_index("s")
    w = ci * 16 + si

    @pl.when(si == 0)
    def _():
        for i in range(13):
            pltpu.sync_copy(z528, acc_r.at[i])

    z16i = jnp.zeros((16,), jnp.int32)

    def zero_body(i, _):
        cntv[pl.ds(16 * i, 16)] = _zero16()
        return _
    lax.fori_loop(0, GPAD // 16, zero_body, None)
    plsc.subcore_barrier()

    ones = jnp.ones((16,), jnp.float32)
    for b in range(WROWS):
        row = w * WROWS + b
        nb = pl.multiple_of(row * 128, 8)

        @pl.when(nb < N)
        def _(row=row, nb=nb):
            pltpu.sync_copy(batch3d.at[row], ibuf)

            @pl.when(nb + 128 <= N)
            def _():
                for i in range(13):
                    pltpu.sync_copy(xs[i].at[pl.ds(nb, 128)], xbuf)
                    pltpu.sync_copy(xbuf, acc_r.at[i].at[ibuf.at[0]], add=True)

            @pl.when(nb + 128 > N)
            def _():
                for i in range(13):
                    pltpu.sync_copy(xs[i].at[pl.ds(nb, N % 128)],
                                    xbuf.at[pl.ds(0, N % 128)])
                    pltpu.sync_copy(xbuf, acc_r.at[i].at[ibuf.at[0]], add=True)

            for g in range(8):
                b16 = ibuf[0, pl.ds(16 * g, 16)]
                plsc.addupdate_scatter(cntv, [b16], ones)

    plsc.subcore_barrier()
    pltpu.sync_copy(cntv, cnt_out.at[w].at[0])

    @pl.when(si < 13)
    def _():
        pltpu.sync_copy(acc_r.at[si], sums_out.at[ci * 13 + si])


def _reduce32_body(dp_ref, cp_ref, do_ref, co_ref):
    @pl.when(pl.program_id(1) == 0)
    def _():
        do_ref[...] = jnp.zeros_like(do_ref)
        co_ref[...] = jnp.zeros_like(co_ref)
    do_ref[...] += dp_ref[0]
    co_ref[...] += cp_ref[0]


def _first_body(nf_ref, cat_ref, degp_ref, cntp_ref, emb_ref, wf_ref, bf_ref,
                w1_ref, x0_ref, dis_ref, y1_ref):
    deg = degp_ref[...] + 1.0
    dis = lax.rsqrt(deg)
    bit = jnp.minimum(cntp_ref[...], 1.0)
    cat = cat_ref[...]
    iota = lax.broadcasted_iota(jnp.int32, (1, VOCAB), 1)
    onehot = (cat == iota).astype(jnp.float32)
    tbl = jnp.dot(emb_ref[...], wf_ref[pl.ds(F_IN, EMB), :],
                  preferred_element_type=jnp.float32)
    x = jnp.dot(nf_ref[...], wf_ref[pl.ds(0, F_IN), :],
                preferred_element_type=jnp.float32)
    x = x + jnp.dot(onehot, tbl, preferred_element_type=jnp.float32)
    x = x + bit * wf_ref[pl.ds(F_IN + EMB, 1), :] + bf_ref[...]
    x0 = jnp.maximum(x, 0.0)
    y = jnp.dot(x0, w1_ref[...], preferred_element_type=jnp.float32) * dis
    x0_ref[...] = x0
    dis_ref[...] = dis
    y1_ref[0] = y[:, :H // 2]
    y1_ref[1] = y[:, H // 2:]


def _layer_body(agg_ref, dis_ref, b_ref, wn_ref, x_ref, yn_ref):
    dis = dis_ref[...]
    a = jnp.concatenate([agg_ref[0], agg_ref[1]], axis=-1)
    x = jnp.maximum(a * dis + b_ref[...], 0.0)
    x_ref[...] = x
    yn = jnp.dot(x, wn_ref[...], preferred_element_type=jnp.float32) * dis
    yn_ref[0] = yn[:, :H // 2]
    yn_ref[1] = yn[:, H // 2:]


def _last_body(agg_ref, dis_ref, b_ref, x_ref):
    a = jnp.concatenate([agg_ref[0], agg_ref[1]], axis=-1)
    x_ref[...] = jnp.maximum(a * dis_ref[...] + b_ref[...], 0.0)


def _div_body(sums_ref, cnt_ref, out_ref):
    counts = jnp.sum(cnt_ref[...], axis=0)[:G]
    cmax = jnp.maximum(counts, 1.0)[:, None]
    for i in range(13):
        si = sums_ref[i, pl.ds(0, G), :] + sums_ref[13 + i, pl.ds(0, G), :]
        out_ref[:, pl.ds(H * i, H)] = si / cmax


_BN = 1000
_NG = N // _BN


def _full(shape):
    return pl.BlockSpec(shape, lambda i: tuple(0 for _ in shape))


def kernel(partial_graph_node_categorical_features, node_features, edge_index,
           edge_features, graph_to_focus_node_map, candidate_attachment_points,
           batch_index, embed_table, W_first, b_first, Ws, bs):
    f32 = jnp.float32
    src = edge_index[0].astype(jnp.int32)
    dst = edge_index[1].astype(jnp.int32)
    ew = edge_features.astype(f32)
    pad = E_PAD - E
    src2d = jnp.pad(src, (0, pad)).reshape(ER, 128)
    dst2d = jnp.pad(dst, (0, pad)).reshape(ER, 128)
    ew2d = jnp.pad(ew, (0, pad)).reshape(ER, 128)
    focus = jnp.concatenate([graph_to_focus_node_map,
                             candidate_attachment_points]).astype(jnp.int32)
    focus3d = focus.reshape(20, 1, 128)
    batch3d = jnp.pad(batch_index.astype(jnp.int32), (0, BR * 128 - N),
                      constant_values=G).reshape(BR, 1, 128)
    z528 = jnp.zeros((GPAD, H), f32)

    deg_p, cnt_p = _stats_kernel(dst2d, ew2d, focus3d)
    deg_p = deg_p.reshape(32, N, 1)
    cnt_p = cnt_p.reshape(32, N, 1)
    deg_r, cnt_r = pl.pallas_call(
        _reduce32_body,
        grid=(_NG, 32),
        in_specs=[
            pl.BlockSpec((1, _BN, 1), lambda i, k: (k, i, 0)),
            pl.BlockSpec((1, _BN, 1), lambda i, k: (k, i, 0)),
        ],
        out_specs=[
            pl.BlockSpec((_BN, 1), lambda i, k: (i, 0)),
            pl.BlockSpec((_BN, 1), lambda i, k: (i, 0)),
        ],
        out_shape=[
            jax.ShapeDtypeStruct((N, 1), f32),
            jax.ShapeDtypeStruct((N, 1), f32),
        ],
    )(deg_p, cnt_p)

    cat2 = partial_graph_node_categorical_features.astype(jnp.int32)[:, None]
    nf = node_features.astype(f32)

    x0, dis, y1 = pl.pallas_call(
        _first_body,
        grid=(_NG,),
        in_specs=[
            pl.BlockSpec((_BN, F_IN), lambda i: (i, 0)),
            pl.BlockSpec((_BN, 1), lambda i: (i, 0)),
            pl.BlockSpec((_BN, 1), lambda i: (i, 0)),
            pl.BlockSpec((_BN, 1), lambda i: (i, 0)),
            _full((VOCAB, EMB)),
            _full((F_IN + EMB + 1, H)),
            _full((1, H)),
            _full((H, H)),
        ],
        out_specs=[
            pl.BlockSpec((_BN, H), lambda i: (i, 0)),
            pl.BlockSpec((_BN, 1), lambda i: (i, 0)),
            pl.BlockSpec((2, _BN, H // 2), lambda i: (0, i, 0)),
        ],
        out_shape=[
            jax.ShapeDtypeStruct((N, H), f32),
            jax.ShapeDtypeStruct((N, 1), f32),
            jax.ShapeDtypeStruct((2, N, H // 2), f32),
        ],
    )(nf, cat2, deg_r, cnt_r, embed_table.astype(f32),
      W_first.astype(f32), b_first.astype(f32)[None, :], Ws[0].astype(f32))

    xs = [x0]
    y = y1
    for i in range(1, L + 1):
        agg = _agg_kernel(y.reshape(2 * N, H // 2), src2d, dst2d, ew2d)
        agg = agg.reshape(2, N, H // 2)
        bias = bs[i - 1].astype(f32)[None, :]
        if i < L:
            xi, y = pl.pallas_call(
                _layer_body,
                grid=(_NG,),
                in_specs=[
                    pl.BlockSpec((2, _BN, H // 2), lambda i: (0, i, 0)),
                    pl.BlockSpec((_BN, 1), lambda i: (i, 0)),
                    _full((1, H)),
                    _full((H, H)),
                ],
                out_specs=[
                    pl.BlockSpec((_BN, H), lambda i: (i, 0)),
                    pl.BlockSpec((2, _BN, H // 2), lambda i: (0, i, 0)),
                ],
                out_shape=[
                    jax.ShapeDtypeStruct((N, H), f32),
                    jax.ShapeDtypeStruct((2, N, H // 2), f32),
                ],
            )(agg, dis, bias, Ws[i].astype(f32))
        else:
            xi = pl.pallas_call(
                _last_body,
                grid=(_NG,),
                in_specs=[
                    pl.BlockSpec((2, _BN, H // 2), lambda i: (0, i, 0)),
                    pl.BlockSpec((_BN, 1), lambda i: (i, 0)),
                    _full((1, H)),
                ],
                out_specs=pl.BlockSpec((_BN, H), lambda i: (i, 0)),
                out_shape=jax.ShapeDtypeStruct((N, H), f32),
            )(agg, dis, bias)
        xs.append(xi)

    sums_p, cnt_parts = _readout_kernel(*xs, batch3d, z528)
    cnt_parts = cnt_parts.reshape(32, GPAD)

    graph_representations = pl.pallas_call(
        _div_body,
        grid=(1,),
        in_specs=[_full((26, GPAD, H)), _full((32, GPAD))],
        out_specs=_full((G, 13 * H)),
        out_shape=jax.ShapeDtypeStruct((G, 13 * H), f32),
    )(sums_p, cnt_parts)

    node_representations = jnp.concatenate(xs, axis=-1)
    return (graph_representations, node_representations)


# speedup vs baseline: 7.1363x; 1.0750x over previous
import functools

import jax
import jax.numpy as jnp
from jax import lax
from jax.experimental import pallas as pl
from jax.experimental.pallas import tpu as pltpu
from jax.experimental.pallas import tpu_sc as plsc

N = 50000
E = 800000
F_IN = 32
EMB = 64
H = 64
L = 12
VOCAB = 139
G = 512
NCAND = 2048

E_PAD = 819200
ER = E_PAD // 128
ROWS_PER_TILE = ER // 16
ROWS_PER_WORKER = ER // 32
NPT = 3128
GPAD = 528
BR = 416
WROWS = 13

_mesh = plsc.VectorSubcoreMesh(core_axis_name="c", subcore_axis_name="s")


def _zero16():
    return jnp.zeros((16,), jnp.float32)


@functools.partial(
    pl.kernel,
    out_type=(jax.ShapeDtypeStruct((32, 1, N), jnp.float32),
              jax.ShapeDtypeStruct((32, 1, N), jnp.float32)),
    mesh=_mesh,
    compiler_params=pltpu.CompilerParams(needs_layout_passes=False, use_tc_tiling_on_sc=False),
    scratch_types=[
        pltpu.VMEM((N,), jnp.float32),
        pltpu.VMEM((N,), jnp.float32),
        pltpu.VMEM((40, 128), jnp.int32),
        pltpu.VMEM((40, 128), jnp.float32),
        pltpu.VMEM((1, 128), jnp.int32),
    ],
)
def _stats_kernel(dst2d, ew2d, focus3d, deg_out, cnt_out,
                  acc_d, acc_f, dbuf, wbuf, fbuf):
    ci = lax.axis_index("c")
    si = lax.axis_index("s")
    w = ci * 16 + si

    z16i = jnp.zeros((16,), jnp.int32)

    def zero_body(i, _):
        acc_d[pl.ds(16 * i, 16)] = _zero16()
        acc_f[pl.ds(16 * i, 16)] = _zero16()
        return _
    lax.fori_loop(0, N // 16, zero_body, None)

    base = w * ROWS_PER_WORKER

    def chunk_body(c, _):
        pltpu.sync_copy(dst2d.at[pl.ds(base + 40 * c, 40)], dbuf)
        pltpu.sync_copy(ew2d.at[pl.ds(base + 40 * c, 40)], wbuf)

        def row_body(j, _):
            for g in range(8):
                d16 = dbuf[j, pl.ds(16 * g, 16)]
                w16 = wbuf[j, pl.ds(16 * g, 16)]
                plsc.addupdate_scatter(acc_d, [d16], w16)
            return _
        lax.fori_loop(0, 40, row_body, None)
        return _
    lax.fori_loop(0, ROWS_PER_WORKER // 40, chunk_body, None)

    @pl.when(w < 20)
    def _():
        pltpu.sync_copy(focus3d.at[w], fbuf)
        ones = jnp.ones((16,), jnp.float32)
        for g in range(8):
            f16 = fbuf[0, pl.ds(16 * g, 16)]
            plsc.addupdate_scatter(acc_f, [f16], ones)

    pltpu.sync_copy(acc_d, deg_out.at[w].at[0])
    pltpu.sync_copy(acc_f, cnt_out.at[w].at[0])


@functools.partial(
    pl.kernel,
    out_type=jax.ShapeDtypeStruct((2 * N, H // 2), jnp.float32),
    mesh=_mesh,
    compiler_params=pltpu.CompilerParams(needs_layout_passes=False, use_tc_tiling_on_sc=False),
    scratch_types=[
        pltpu.VMEM((2, 8, 128), jnp.int32),
        pltpu.VMEM((2, 8, 128), jnp.int32),
        pltpu.VMEM((2, 8, 128), jnp.float32),
        pltpu.VMEM((4, 128, H // 2), jnp.float32),
        pltpu.VMEM_SHARED((N, H // 2), jnp.float32),
        pltpu.SemaphoreType.DMA((4,)),
        pltpu.SemaphoreType.DMA((4,)),
    ],
)
def _agg_kernel(y_cat, src2d, dst2d, ew2d, agg_out,
                sbuf, dbuf, wbuf, gbuf, acc_s, sem_g, sem_s):
    ci = lax.axis_index("c")
    si = lax.axis_index("s")
    coff = ci * N

    nbase = pl.multiple_of(jnp.where(si == 15, N - NPT, NPT * si), 8)
    pltpu.sync_copy(y_cat.at[pl.ds(coff + nbase, NPT)],
                    acc_s.at[pl.ds(nbase, NPT)])
    plsc.subcore_barrier()

    ebase = si * ROWS_PER_TILE
    off16 = jnp.full((16,), coff, jnp.int32)
    RPT = ROWS_PER_TILE

    def idx_load(r0, q):
        pltpu.sync_copy(src2d.at[pl.ds(r0, 8)], sbuf.at[q])
        pltpu.sync_copy(dst2d.at[pl.ds(r0, 8)], dbuf.at[q])
        pltpu.sync_copy(ew2d.at[pl.ds(r0, 8)], wbuf.at[q])
        for jr in range(8):
            for g in range(8):
                sl = pl.ds(16 * g, 16)
                sbuf[q, jr, sl] = sbuf[q, jr, sl] + off16

    def g_issue(q, jr, slot):
        pltpu.async_copy(y_cat.at[sbuf.at[q].at[jr]], gbuf.at[slot],
                         sem_g.at[slot])

    def g_wait(slot):
        pltpu.make_async_copy(y_cat.at[sbuf.at[0].at[0]], gbuf.at[slot],
                              sem_g.at[slot]).wait()

    def s_issue(q, jr, slot):
        pltpu.async_copy(gbuf.at[slot], acc_s.at[dbuf.at[q].at[jr]],
                         sem_s.at[slot], add=True)

    def s_drain(slot):
        pltpu.make_async_copy(gbuf.at[slot], acc_s.at[dbuf.at[0].at[0]],
                              sem_s.at[slot]).wait()

    def scale(q, jr, slot):
        def scale_body(i, _):
            for u4 in range(4):
                e = 4 * i + u4
                w16 = plsc.load_gather(
                    wbuf.at[q], [jnp.full((16,), jr, jnp.int32),
                                 jnp.full((16,), e, jnp.int32)])
                gbuf[slot, e, pl.ds(0, 16)] = \
                    gbuf[slot, e, pl.ds(0, 16)] * w16
                gbuf[slot, e, pl.ds(16, 16)] = \
                    gbuf[slot, e, pl.ds(16, 16)] * w16
            return _
        lax.fori_loop(0, 32, scale_body, None)

    idx_load(ebase, 0)
    g_issue(0, 0, 0)
    g_issue(0, 1, 1)

    def block_body(it, _):
        for u in range(16):
            m = 16 * it + u
            slot = u % 4
            nslot = (u + 2) % 4
            q = (u // 8) % 2
            qn = ((u + 2) // 8) % 2
            jr = u % 8

            @pl.when(m >= 2)
            def _():
                s_drain(nslot)
            if u in (6, 14):
                @pl.when(m + 2 < RPT)
                def _(m=m):
                    idx_load(pl.multiple_of(ebase + m + 2, 8), (u // 8 + 1) % 2)

            @pl.when(m + 2 < RPT)
            def _():
                g_issue(qn, (u + 2) % 8, nslot)
            g_wait(slot)
            scale(q, jr, slot)
            s_issue(q, jr, slot)
        return _
    lax.fori_loop(0, RPT // 16, block_body, None)
    for slot in (2, 3):
        s_drain(slot)

    plsc.subcore_barrier()
    pltpu.sync_copy(acc_s.at[pl.ds(nbase, NPT)],
                    agg_out.at[pl.ds(coff + nbase, NPT)])


@functools.partial(
    pl.kernel,
    out_type=(jax.ShapeDtypeStruct((26, GPAD, H), jnp.float32),
              jax.ShapeDtypeStruct((32, 1, GPAD), jnp.float32)),
    mesh=_mesh,
    compiler_params=pltpu.CompilerParams(needs_layout_passes=False, use_tc_tiling_on_sc=False),
    scratch_types=[
        pltpu.VMEM((128, H), jnp.float32),
        pltpu.VMEM((1, 128), jnp.int32),
        pltpu.VMEM((GPAD,), jnp.float32),
        pltpu.VMEM_SHARED((13, GPAD, H), jnp.float32),
    ],
)
def _readout_kernel(*args):
    xs = args[:13]
    batch3d, z528 = args[13], args[14]
    sums_out, cnt_out = args[15], args[16]
    xbuf, ibuf, cntv, acc_r = args[17:]

    ci = lax.axis_index("c")
    si = lax.axis_index("s")
    w = ci * 16 + si

    @pl.when(si == 0)
    def _():
        for i in range(13):
            pltpu.sync_copy(z528, acc_r.at[i])

    z16i = jnp.zeros((16,), jnp.int32)

    def zero_body(i, _):
        cntv[pl.ds(16 * i, 16)] = _zero16()
        return _
    lax.fori_loop(0, GPAD // 16, zero_body, None)
    plsc.subcore_barrier()

    ones = jnp.ones((16,), jnp.float32)
    for b in range(WROWS):
        row = w * WROWS + b
        nb = pl.multiple_of(row * 128, 8)

        @pl.when(nb < N)
        def _(row=row, nb=nb):
            pltpu.sync_copy(batch3d.at[row], ibuf)

            @pl.when(nb + 128 <= N)
            def _():
                for i in range(13):
                    pltpu.sync_copy(xs[i].at[pl.ds(nb, 128)], xbuf)
                    pltpu.sync_copy(xbuf, acc_r.at[i].at[ibuf.at[0]], add=True)

            @pl.when(nb + 128 > N)
            def _():
                for i in range(13):
                    pltpu.sync_copy(xs[i].at[pl.ds(nb, N % 128)],
                                    xbuf.at[pl.ds(0, N % 128)])
                    pltpu.sync_copy(xbuf, acc_r.at[i].at[ibuf.at[0]], add=True)

            for g in range(8):
                b16 = ibuf[0, pl.ds(16 * g, 16)]
                plsc.addupdate_scatter(cntv, [b16], ones)

    plsc.subcore_barrier()
    pltpu.sync_copy(cntv, cnt_out.at[w].at[0])

    @pl.when(si < 13)
    def _():
        pltpu.sync_copy(acc_r.at[si], sums_out.at[ci * 13 + si])


def _reduce32_body(dp_ref, cp_ref, do_ref, co_ref):
    @pl.when(pl.program_id(1) == 0)
    def _():
        do_ref[...] = jnp.zeros_like(do_ref)
        co_ref[...] = jnp.zeros_like(co_ref)
    do_ref[...] += dp_ref[0]
    co_ref[...] += cp_ref[0]


def _first_body(nf_ref, cat_ref, degp_ref, cntp_ref, emb_ref, wf_ref, bf_ref,
                w1_ref, x0_ref, dis_ref, y1_ref):
    deg = degp_ref[...] + 1.0
    dis = lax.rsqrt(deg)
    bit = jnp.minimum(cntp_ref[...], 1.0)
    cat = cat_ref[...]
    iota = lax.broadcasted_iota(jnp.int32, (1, VOCAB), 1)
    onehot = (cat == iota).astype(jnp.float32)
    tbl = jnp.dot(emb_ref[...], wf_ref[pl.ds(F_IN, EMB), :],
                  preferred_element_type=jnp.float32)
    x = jnp.dot(nf_ref[...], wf_ref[pl.ds(0, F_IN), :],
                preferred_element_type=jnp.float32)
    x = x + jnp.dot(onehot, tbl, preferred_element_type=jnp.float32)
    x = x + bit * wf_ref[pl.ds(F_IN + EMB, 1), :] + bf_ref[...]
    x0 = jnp.maximum(x, 0.0)
    y = jnp.dot(x0, w1_ref[...], preferred_element_type=jnp.float32) * dis
    x0_ref[...] = x0
    dis_ref[...] = dis
    y1_ref[0] = y[:, :H // 2]
    y1_ref[1] = y[:, H // 2:]


def _layer_body(agg_ref, dis_ref, b_ref, wn_ref, x_ref, yn_ref):
    dis = dis_ref[...]
    a = jnp.concatenate([agg_ref[0], agg_ref[1]], axis=-1)
    x = jnp.maximum(a * dis + b_ref[...], 0.0)
    x_ref[...] = x
    yn = jnp.dot(x, wn_ref[...], preferred_element_type=jnp.float32) * dis
    yn_ref[0] = yn[:, :H // 2]
    yn_ref[1] = yn[:, H // 2:]


def _last_body(agg_ref, dis_ref, b_ref, x_ref):
    a = jnp.concatenate([agg_ref[0], agg_ref[1]], axis=-1)
    x_ref[...] = jnp.maximum(a * dis_ref[...] + b_ref[...], 0.0)


def _concat_body(*refs):
    out_ref = refs[-1]
    for i in range(13):
        out_ref[:, H * i:H * (i + 1)] = refs[i][...]


def _div_body(sums_ref, cnt_ref, out_ref):
    counts = jnp.sum(cnt_ref[...], axis=0)[:G]
    cmax = jnp.maximum(counts, 1.0)[:, None]
    for i in range(13):
        si = sums_ref[i, pl.ds(0, G), :] + sums_ref[13 + i, pl.ds(0, G), :]
        out_ref[:, pl.ds(H * i, H)] = si / cmax


_BN = 1000
_NG = N // _BN


def _full(shape):
    return pl.BlockSpec(shape, lambda i: tuple(0 for _ in shape))


def kernel(partial_graph_node_categorical_features, node_features, edge_index,
           edge_features, graph_to_focus_node_map, candidate_attachment_points,
           batch_index, embed_table, W_first, b_first, Ws, bs):
    f32 = jnp.float32
    src = edge_index[0].astype(jnp.int32)
    dst = edge_index[1].astype(jnp.int32)
    ew = edge_features.astype(f32)
    pad = E_PAD - E
    src2d = jnp.pad(src, (0, pad)).reshape(ER, 128)
    dst2d = jnp.pad(dst, (0, pad)).reshape(ER, 128)
    ew2d = jnp.pad(ew, (0, pad)).reshape(ER, 128)
    focus = jnp.concatenate([graph_to_focus_node_map,
                             candidate_attachment_points]).astype(jnp.int32)
    focus3d = focus.reshape(20, 1, 128)
    batch3d = jnp.pad(batch_index.astype(jnp.int32), (0, BR * 128 - N),
                      constant_values=G).reshape(BR, 1, 128)
    z528 = jnp.zeros((GPAD, H), f32)

    deg_p, cnt_p = _stats_kernel(dst2d, ew2d, focus3d)
    deg_p = deg_p.reshape(32, N, 1)
    cnt_p = cnt_p.reshape(32, N, 1)
    deg_r, cnt_r = pl.pallas_call(
        _reduce32_body,
        grid=(_NG, 32),
        in_specs=[
            pl.BlockSpec((1, _BN, 1), lambda i, k: (k, i, 0)),
            pl.BlockSpec((1, _BN, 1), lambda i, k: (k, i, 0)),
        ],
        out_specs=[
            pl.BlockSpec((_BN, 1), lambda i, k: (i, 0)),
            pl.BlockSpec((_BN, 1), lambda i, k: (i, 0)),
        ],
        out_shape=[
            jax.ShapeDtypeStruct((N, 1), f32),
            jax.ShapeDtypeStruct((N, 1), f32),
        ],
    )(deg_p, cnt_p)

    cat2 = partial_graph_node_categorical_features.astype(jnp.int32)[:, None]
    nf = node_features.astype(f32)

    x0, dis, y1 = pl.pallas_call(
        _first_body,
        grid=(_NG,),
        in_specs=[
            pl.BlockSpec((_BN, F_IN), lambda i: (i, 0)),
            pl.BlockSpec((_BN, 1), lambda i: (i, 0)),
            pl.BlockSpec((_BN, 1), lambda i: (i, 0)),
            pl.BlockSpec((_BN, 1), lambda i: (i, 0)),
            _full((VOCAB, EMB)),
            _full((F_IN + EMB + 1, H)),
            _full((1, H)),
            _full((H, H)),
        ],
        out_specs=[
            pl.BlockSpec((_BN, H), lambda i: (i, 0)),
            pl.BlockSpec((_BN, 1), lambda i: (i, 0)),
            pl.BlockSpec((2, _BN, H // 2), lambda i: (0, i, 0)),
        ],
        out_shape=[
            jax.ShapeDtypeStruct((N, H), f32),
            jax.ShapeDtypeStruct((N, 1), f32),
            jax.ShapeDtypeStruct((2, N, H // 2), f32),
        ],
    )(nf, cat2, deg_r, cnt_r, embed_table.astype(f32),
      W_first.astype(f32), b_first.astype(f32)[None, :], Ws[0].astype(f32))

    xs = [x0]
    y = y1
    for i in range(1, L + 1):
        agg = _agg_kernel(y.reshape(2 * N, H // 2), src2d, dst2d, ew2d)
        agg = agg.reshape(2, N, H // 2)
        bias = bs[i - 1].astype(f32)[None, :]
        if i < L:
            xi, y = pl.pallas_call(
                _layer_body,
                grid=(_NG,),
                in_specs=[
                    pl.BlockSpec((2, _BN, H // 2), lambda i: (0, i, 0)),
                    pl.BlockSpec((_BN, 1), lambda i: (i, 0)),
                    _full((1, H)),
                    _full((H, H)),
                ],
                out_specs=[
                    pl.BlockSpec((_BN, H), lambda i: (i, 0)),
                    pl.BlockSpec((2, _BN, H // 2), lambda i: (0, i, 0)),
                ],
                out_shape=[
                    jax.ShapeDtypeStruct((N, H), f32),
                    jax.ShapeDtypeStruct((2, N, H // 2), f32),
                ],
            )(agg, dis, bias, Ws[i].astype(f32))
        else:
            xi = pl.pallas_call(
                _last_body,
                grid=(_NG,),
                in_specs=[
                    pl.BlockSpec((2, _BN, H // 2), lambda i: (0, i, 0)),
                    pl.BlockSpec((_BN, 1), lambda i: (i, 0)),
                    _full((1, H)),
                ],
                out_specs=pl.BlockSpec((_BN, H), lambda i: (i, 0)),
                out_shape=jax.ShapeDtypeStruct((N, H), f32),
            )(agg, dis, bias)
        xs.append(xi)

    sums_p, cnt_parts = _readout_kernel(*xs, batch3d, z528)
    cnt_parts = cnt_parts.reshape(32, GPAD)

    graph_representations = pl.pallas_call(
        _div_body,
        grid=(1,),
        in_specs=[_full((26, GPAD, H)), _full((32, GPAD))],
        out_specs=_full((G, 13 * H)),
        out_shape=jax.ShapeDtypeStruct((G, 13 * H), f32),
    )(sums_p, cnt_parts)

    node_representations = pl.pallas_call(
        _concat_body,
        grid=(_NG,),
        in_specs=[pl.BlockSpec((_BN, H), lambda i: (i, 0))] * 13,
        out_specs=pl.BlockSpec((_BN, 13 * H), lambda i: (i, 0)),
        out_shape=jax.ShapeDtypeStruct((N, 13 * H), f32),
    )(*xs)
    return (graph_representations, node_representations)


# speedup vs baseline: 7.4604x; 1.0454x over previous
import functools

import jax
import jax.numpy as jnp
from jax import lax
from jax.experimental import pallas as pl
from jax.experimental.pallas import tpu as pltpu
from jax.experimental.pallas import tpu_sc as plsc

N = 50000
E = 800000
F_IN = 32
EMB = 64
H = 64
L = 12
VOCAB = 139
G = 512
NCAND = 2048

E_PAD = 819200
ER = E_PAD // 128
ROWS_PER_TILE = ER // 16
ROWS_PER_WORKER = ER // 32
NPT = 3128
GPAD = 528
BR = 416
WROWS = 13

_mesh = plsc.VectorSubcoreMesh(core_axis_name="c", subcore_axis_name="s")


def _zero16():
    return jnp.zeros((16,), jnp.float32)


@functools.partial(
    pl.kernel,
    out_type=(jax.ShapeDtypeStruct((32, 1, N), jnp.float32),
              jax.ShapeDtypeStruct((32, 1, N), jnp.float32)),
    mesh=_mesh,
    compiler_params=pltpu.CompilerParams(needs_layout_passes=False, use_tc_tiling_on_sc=False),
    scratch_types=[
        pltpu.VMEM((N,), jnp.float32),
        pltpu.VMEM((N,), jnp.float32),
        pltpu.VMEM((40, 128), jnp.int32),
        pltpu.VMEM((40, 128), jnp.float32),
        pltpu.VMEM((1, 128), jnp.int32),
    ],
)
def _stats_kernel(dst2d, ew2d, focus3d, deg_out, cnt_out,
                  acc_d, acc_f, dbuf, wbuf, fbuf):
    ci = lax.axis_index("c")
    si = lax.axis_index("s")
    w = ci * 16 + si

    z16i = jnp.zeros((16,), jnp.int32)

    def zero_body(i, _):
        acc_d[pl.ds(16 * i, 16)] = _zero16()
        acc_f[pl.ds(16 * i, 16)] = _zero16()
        return _
    lax.fori_loop(0, N // 16, zero_body, None)

    base = w * ROWS_PER_WORKER

    def chunk_body(c, _):
        pltpu.sync_copy(dst2d.at[pl.ds(base + 40 * c, 40)], dbuf)
        pltpu.sync_copy(ew2d.at[pl.ds(base + 40 * c, 40)], wbuf)

        def row_body(j, _):
            for g in range(8):
                d16 = dbuf[j, pl.ds(16 * g, 16)]
                w16 = wbuf[j, pl.ds(16 * g, 16)]
                plsc.addupdate_scatter(acc_d, [d16], w16)
            return _
        lax.fori_loop(0, 40, row_body, None)
        return _
    lax.fori_loop(0, ROWS_PER_WORKER // 40, chunk_body, None)

    @pl.when(w < 20)
    def _():
        pltpu.sync_copy(focus3d.at[w], fbuf)
        ones = jnp.ones((16,), jnp.float32)
        for g in range(8):
            f16 = fbuf[0, pl.ds(16 * g, 16)]
            plsc.addupdate_scatter(acc_f, [f16], ones)

    pltpu.sync_copy(acc_d, deg_out.at[w].at[0])
    pltpu.sync_copy(acc_f, cnt_out.at[w].at[0])


@functools.partial(
    pl.kernel,
    out_type=jax.ShapeDtypeStruct((2 * N, H // 2), jnp.float32),
    mesh=_mesh,
    compiler_params=pltpu.CompilerParams(needs_layout_passes=False, use_tc_tiling_on_sc=False),
    scratch_types=[
        pltpu.VMEM((2, 8, 128), jnp.int32),
        pltpu.VMEM((2, 8, 128), jnp.int32),
        pltpu.VMEM((2, 8, 128), jnp.float32),
        pltpu.VMEM((4, 128, H // 2), jnp.float32),
        pltpu.VMEM_SHARED((N, H // 2), jnp.float32),
        pltpu.SemaphoreType.DMA((4,)),
        pltpu.SemaphoreType.DMA((4,)),
        pltpu.SemaphoreType.DMA((2,)),
    ],
)
def _agg_kernel(y_cat, src2d, dst2d, ew2d, agg_out,
                sbuf, dbuf, wbuf, gbuf, acc_s, sem_g, sem_s, sem_i):
    ci = lax.axis_index("c")
    si = lax.axis_index("s")
    coff = ci * N

    nbase = pl.multiple_of(jnp.where(si == 15, N - NPT, NPT * si), 8)
    pltpu.sync_copy(y_cat.at[pl.ds(coff + nbase, NPT)],
                    acc_s.at[pl.ds(nbase, NPT)])
    plsc.subcore_barrier()

    ebase = si * ROWS_PER_TILE
    off16 = jnp.full((16,), coff, jnp.int32)
    RPT = ROWS_PER_TILE

    def idx_issue(r0, q):
        pltpu.async_copy(src2d.at[pl.ds(r0, 8)], sbuf.at[q], sem_i.at[q])
        pltpu.async_copy(dst2d.at[pl.ds(r0, 8)], dbuf.at[q], sem_i.at[q])
        pltpu.async_copy(ew2d.at[pl.ds(r0, 8)], wbuf.at[q], sem_i.at[q])

    def idx_wait(q):
        pltpu.make_async_copy(src2d.at[pl.ds(0, 8)], sbuf.at[q],
                              sem_i.at[q]).wait()
        pltpu.make_async_copy(dst2d.at[pl.ds(0, 8)], dbuf.at[q],
                              sem_i.at[q]).wait()
        pltpu.make_async_copy(ew2d.at[pl.ds(0, 8)], wbuf.at[q],
                              sem_i.at[q]).wait()
        for jr in range(8):
            for g in range(8):
                sl = pl.ds(16 * g, 16)
                sbuf[q, jr, sl] = sbuf[q, jr, sl] + off16

    def g_issue(q, jr, slot):
        pltpu.async_copy(y_cat.at[sbuf.at[q].at[jr]], gbuf.at[slot],
                         sem_g.at[slot])

    def g_wait(slot):
        pltpu.make_async_copy(y_cat.at[sbuf.at[0].at[0]], gbuf.at[slot],
                              sem_g.at[slot]).wait()

    def s_issue(q, jr, slot):
        pltpu.async_copy(gbuf.at[slot], acc_s.at[dbuf.at[q].at[jr]],
                         sem_s.at[slot], add=True)

    def s_drain(slot):
        pltpu.make_async_copy(gbuf.at[slot], acc_s.at[dbuf.at[0].at[0]],
                              sem_s.at[slot]).wait()

    def scale(q, jr, slot):
        def scale_body(i, _):
            for u4 in range(8):
                e = 8 * i + u4
                w16 = plsc.load_gather(
                    wbuf.at[q], [jnp.full((16,), jr, jnp.int32),
                                 jnp.full((16,), e, jnp.int32)])
                gbuf[slot, e, pl.ds(0, 16)] = \
                    gbuf[slot, e, pl.ds(0, 16)] * w16
                gbuf[slot, e, pl.ds(16, 16)] = \
                    gbuf[slot, e, pl.ds(16, 16)] * w16
            return _
        lax.fori_loop(0, 16, scale_body, None)

    idx_issue(ebase, 0)
    idx_wait(0)
    g_issue(0, 0, 0)
    g_issue(0, 1, 1)

    def block_body(it, _):
        for u in range(16):
            m = 16 * it + u
            slot = u % 4
            nslot = (u + 2) % 4
            q = (u // 8) % 2
            qn = ((u + 2) // 8) % 2
            jr = u % 8

            @pl.when(m >= 2)
            def _():
                s_drain(nslot)
            if u in (5, 13):
                @pl.when(m + 3 < RPT)
                def _(m=m):
                    idx_issue(pl.multiple_of(ebase + m + 3, 8),
                              (u // 8 + 1) % 2)
            if u in (6, 14):
                @pl.when(m + 2 < RPT)
                def _():
                    idx_wait((u // 8 + 1) % 2)

            @pl.when(m + 2 < RPT)
            def _():
                g_issue(qn, (u + 2) % 8, nslot)
            g_wait(slot)
            scale(q, jr, slot)
            s_issue(q, jr, slot)
        return _
    lax.fori_loop(0, RPT // 16, block_body, None)
    for slot in (2, 3):
        s_drain(slot)

    plsc.subcore_barrier()
    pltpu.sync_copy(acc_s.at[pl.ds(nbase, NPT)],
                    agg_out.at[pl.ds(coff + nbase, NPT)])


@functools.partial(
    pl.kernel,
    out_type=(jax.ShapeDtypeStruct((26, GPAD, H), jnp.float32),
              jax.ShapeDtypeStruct((32, 1, GPAD), jnp.float32)),
    mesh=_mesh,
    compiler_params=pltpu.CompilerParams(needs_layout_passes=False, use_tc_tiling_on_sc=False),
    scratch_types=[
        pltpu.VMEM((128, H), jnp.float32),
        pltpu.VMEM((1, 128), jnp.int32),
        pltpu.VMEM((GPAD,), jnp.float32),
        pltpu.VMEM_SHARED((13, GPAD, H), jnp.float32),
    ],
)
def _readout_kernel(*args):
    xs = args[:13]
    batch3d, z528 = args[13], args[14]
    sums_out, cnt_out = args[15], args[16]
    xbuf, ibuf, cntv, acc_r = args[17:]

    ci = lax.axis_index("c")
    si = lax.axis_index("s")
    w = ci * 16 + si

    @pl.when(si == 0)
    def _():
        for i in range(13):
            pltpu.sync_copy(z528, acc_r.at[i])

    z16i = jnp.zeros((16,), jnp.int32)

    def zero_body(i, _):
        cntv[pl.ds(16 * i, 16)] = _zero16()
        return _
    lax.fori_loop(0, GPAD // 16, zero_body, None)
    plsc.subcore_barrier()

    ones = jnp.ones((16,), jnp.float32)
    for b in range(WROWS):
        row = w * WROWS + b
        nb = pl.multiple_of(row * 128, 8)

        @pl.when(nb < N)
        def _(row=row, nb=nb):
            pltpu.sync_copy(batch3d.at[row], ibuf)

            @pl.when(nb + 128 <= N)
            def _():
                for i in range(13):
                    pltpu.sync_copy(xs[i].at[pl.ds(nb, 128)], xbuf)
                    pltpu.sync_copy(xbuf, acc_r.at[i].at[ibuf.at[0]], add=True)

            @pl.when(nb + 128 > N)
            def _():
                for i in range(13):
                    pltpu.sync_copy(xs[i].at[pl.ds(nb, N % 128)],
                                    xbuf.at[pl.ds(0, N % 128)])
                    pltpu.sync_copy(xbuf, acc_r.at[i].at[ibuf.at[0]], add=True)

            for g in range(8):
                b16 = ibuf[0, pl.ds(16 * g, 16)]
                plsc.addupdate_scatter(cntv, [b16], ones)

    plsc.subcore_barrier()
    pltpu.sync_copy(cntv, cnt_out.at[w].at[0])

    @pl.when(si < 13)
    def _():
        pltpu.sync_copy(acc_r.at[si], sums_out.at[ci * 13 + si])


def _reduce32_body(dp_ref, cp_ref, do_ref, co_ref):
    @pl.when(pl.program_id(1) == 0)
    def _():
        do_ref[...] = jnp.zeros_like(do_ref)
        co_ref[...] = jnp.zeros_like(co_ref)
    do_ref[...] += dp_ref[0]
    co_ref[...] += cp_ref[0]


def _first_body(nf_ref, cat_ref, degp_ref, cntp_ref, emb_ref, wf_ref, bf_ref,
                w1_ref, x0_ref, dis_ref, y1_ref):
    deg = degp_ref[...] + 1.0
    dis = lax.rsqrt(deg)
    bit = jnp.minimum(cntp_ref[...], 1.0)
    cat = cat_ref[...]
    iota = lax.broadcasted_iota(jnp.int32, (1, VOCAB), 1)
    onehot = (cat == iota).astype(jnp.float32)
    tbl = jnp.dot(emb_ref[...], wf_ref[pl.ds(F_IN, EMB), :],
                  preferred_element_type=jnp.float32)
    x = jnp.dot(nf_ref[...], wf_ref[pl.ds(0, F_IN), :],
                preferred_element_type=jnp.float32)
    x = x + jnp.dot(onehot, tbl, preferred_element_type=jnp.float32)
    x = x + bit * wf_ref[pl.ds(F_IN + EMB, 1), :] + bf_ref[...]
    x0 = jnp.maximum(x, 0.0)
    y = jnp.dot(x0, w1_ref[...], preferred_element_type=jnp.float32) * dis
    x0_ref[...] = x0
    dis_ref[...] = dis
    y1_ref[0] = y[:, :H // 2]
    y1_ref[1] = y[:, H // 2:]


def _layer_body(agg_ref, dis_ref, b_ref, wn_ref, x_ref, yn_ref):
    dis = dis_ref[...]
    a = jnp.concatenate([agg_ref[0], agg_ref[1]], axis=-1)
    x = jnp.maximum(a * dis + b_ref[...], 0.0)
    x_ref[...] = x
    yn = jnp.dot(x, wn_ref[...], preferred_element_type=jnp.float32) * dis
    yn_ref[0] = yn[:, :H // 2]
    yn_ref[1] = yn[:, H // 2:]


def _last_body(agg_ref, dis_ref, b_ref, x_ref):
    a = jnp.concatenate([agg_ref[0], agg_ref[1]], axis=-1)
    x_ref[...] = jnp.maximum(a * dis_ref[...] + b_ref[...], 0.0)


def _concat_body(*refs):
    out_ref = refs[-1]
    for i in range(13):
        out_ref[:, H * i:H * (i + 1)] = refs[i][...]


def _div_body(sums_ref, cnt_ref, out_ref):
    counts = jnp.sum(cnt_ref[...], axis=0)[:G]
    cmax = jnp.maximum(counts, 1.0)[:, None]
    for i in range(13):
        si = sums_ref[i, pl.ds(0, G), :] + sums_ref[13 + i, pl.ds(0, G), :]
        out_ref[:, pl.ds(H * i, H)] = si / cmax


_BN = 1000
_NG = N // _BN


def _full(shape):
    return pl.BlockSpec(shape, lambda i: tuple(0 for _ in shape))


def kernel(partial_graph_node_categorical_features, node_features, edge_index,
           edge_features, graph_to_focus_node_map, candidate_attachment_points,
           batch_index, embed_table, W_first, b_first, Ws, bs):
    f32 = jnp.float32
    src = edge_index[0].astype(jnp.int32)
    dst = edge_index[1].astype(jnp.int32)
    ew = edge_features.astype(f32)
    pad = E_PAD - E
    src2d = jnp.pad(src, (0, pad)).reshape(ER, 128)
    dst2d = jnp.pad(dst, (0, pad)).reshape(ER, 128)
    ew2d = jnp.pad(ew, (0, pad)).reshape(ER, 128)
    focus = jnp.concatenate([graph_to_focus_node_map,
                             candidate_attachment_points]).astype(jnp.int32)
    focus3d = focus.reshape(20, 1, 128)
    batch3d = jnp.pad(batch_index.astype(jnp.int32), (0, BR * 128 - N),
                      constant_values=G).reshape(BR, 1, 128)
    z528 = jnp.zeros((GPAD, H), f32)

    deg_p, cnt_p = _stats_kernel(dst2d, ew2d, focus3d)
    deg_p = deg_p.reshape(32, N, 1)
    cnt_p = cnt_p.reshape(32, N, 1)
    deg_r, cnt_r = pl.pallas_call(
        _reduce32_body,
        grid=(_NG, 32),
        in_specs=[
            pl.BlockSpec((1, _BN, 1), lambda i, k: (k, i, 0)),
            pl.BlockSpec((1, _BN, 1), lambda i, k: (k, i, 0)),
        ],
        out_specs=[
            pl.BlockSpec((_BN, 1), lambda i, k: (i, 0)),
            pl.BlockSpec((_BN, 1), lambda i, k: (i, 0)),
        ],
        out_shape=[
            jax.ShapeDtypeStruct((N, 1), f32),
            jax.ShapeDtypeStruct((N, 1), f32),
        ],
    )(deg_p, cnt_p)

    cat2 = partial_graph_node_categorical_features.astype(jnp.int32)[:, None]
    nf = node_features.astype(f32)

    x0, dis, y1 = pl.pallas_call(
        _first_body,
        grid=(_NG,),
        in_specs=[
            pl.BlockSpec((_BN, F_IN), lambda i: (i, 0)),
            pl.BlockSpec((_BN, 1), lambda i: (i, 0)),
            pl.BlockSpec((_BN, 1), lambda i: (i, 0)),
            pl.BlockSpec((_BN, 1), lambda i: (i, 0)),
            _full((VOCAB, EMB)),
            _full((F_IN + EMB + 1, H)),
            _full((1, H)),
            _full((H, H)),
        ],
        out_specs=[
            pl.BlockSpec((_BN, H), lambda i: (i, 0)),
            pl.BlockSpec((_BN, 1), lambda i: (i, 0)),
            pl.BlockSpec((2, _BN, H // 2), lambda i: (0, i, 0)),
        ],
        out_shape=[
            jax.ShapeDtypeStruct((N, H), f32),
            jax.ShapeDtypeStruct((N, 1), f32),
            jax.ShapeDtypeStruct((2, N, H // 2), f32),
        ],
    )(nf, cat2, deg_r, cnt_r, embed_table.astype(f32),
      W_first.astype(f32), b_first.astype(f32)[None, :], Ws[0].astype(f32))

    xs = [x0]
    y = y1
    for i in range(1, L + 1):
        agg = _agg_kernel(y.reshape(2 * N, H // 2), src2d, dst2d, ew2d)
        agg = agg.reshape(2, N, H // 2)
        bias = bs[i - 1].astype(f32)[None, :]
        if i < L:
            xi, y = pl.pallas_call(
                _layer_body,
                grid=(_NG,),
                in_specs=[
                    pl.BlockSpec((2, _BN, H // 2), lambda i: (0, i, 0)),
                    pl.BlockSpec((_BN, 1), lambda i: (i, 0)),
                    _full((1, H)),
                    _full((H, H)),
                ],
                out_specs=[
                    pl.BlockSpec((_BN, H), lambda i: (i, 0)),
                    pl.BlockSpec((2, _BN, H // 2), lambda i: (0, i, 0)),
                ],
                out_shape=[
                    jax.ShapeDtypeStruct((N, H), f32),
                    jax.ShapeDtypeStruct((2, N, H // 2), f32),
                ],
            )(agg, dis, bias, Ws[i].astype(f32))
        else:
            xi = pl.pallas_call(
                _last_body,
                grid=(_NG,),
                in_specs=[
                    pl.BlockSpec((2, _BN, H // 2), lambda i: (0, i, 0)),
                    pl.BlockSpec((_BN, 1), lambda i: (i, 0)),
                    _full((1, H)),
                ],
                out_specs=pl.BlockSpec((_BN, H), lambda i: (i, 0)),
                out_shape=jax.ShapeDtypeStruct((N, H), f32),
            )(agg, dis, bias)
        xs.append(xi)

    sums_p, cnt_parts = _readout_kernel(*xs, batch3d, z528)
    cnt_parts = cnt_parts.reshape(32, GPAD)

    graph_representations = pl.pallas_call(
        _div_body,
        grid=(1,),
        in_specs=[_full((26, GPAD, H)), _full((32, GPAD))],
        out_specs=_full((G, 13 * H)),
        out_shape=jax.ShapeDtypeStruct((G, 13 * H), f32),
    )(sums_p, cnt_parts)

    node_representations = pl.pallas_call(
        _concat_body,
        grid=(_NG,),
        in_specs=[pl.BlockSpec((_BN, H), lambda i: (i, 0))] * 13,
        out_specs=pl.BlockSpec((_BN, 13 * H), lambda i: (i, 0)),
        out_shape=jax.ShapeDtypeStruct((N, 13 * H), f32),
    )(*xs)
    return (graph_representations, node_representations)


# speedup vs baseline: 7.5092x; 1.0065x over previous
import functools

import jax
import jax.numpy as jnp
from jax import lax
from jax.experimental import pallas as pl
from jax.experimental.pallas import tpu as pltpu
from jax.experimental.pallas import tpu_sc as plsc

N = 50000
E = 800000
F_IN = 32
EMB = 64
H = 64
L = 12
VOCAB = 139
G = 512
NCAND = 2048

E_PAD = 819200
ER = E_PAD // 128
ROWS_PER_TILE = ER // 16
ROWS_PER_WORKER = ER // 32
NPT = 3128
GPAD = 528
BR = 416
WROWS = 13

_mesh = plsc.VectorSubcoreMesh(core_axis_name="c", subcore_axis_name="s")


def _zero16():
    return jnp.zeros((16,), jnp.float32)


@functools.partial(
    pl.kernel,
    out_type=(jax.ShapeDtypeStruct((32, 1, N), jnp.float32),
              jax.ShapeDtypeStruct((32, 1, N), jnp.float32)),
    mesh=_mesh,
    compiler_params=pltpu.CompilerParams(needs_layout_passes=False, use_tc_tiling_on_sc=False),
    scratch_types=[
        pltpu.VMEM((N,), jnp.float32),
        pltpu.VMEM((N,), jnp.float32),
        pltpu.VMEM((40, 128), jnp.int32),
        pltpu.VMEM((40, 128), jnp.float32),
        pltpu.VMEM((1, 128), jnp.int32),
    ],
)
def _stats_kernel(dst2d, ew2d, focus3d, deg_out, cnt_out,
                  acc_d, acc_f, dbuf, wbuf, fbuf):
    ci = lax.axis_index("c")
    si = lax.axis_index("s")
    w = ci * 16 + si

    z16i = jnp.zeros((16,), jnp.int32)

    def zero_body(i, _):
        acc_d[pl.ds(16 * i, 16)] = _zero16()
        acc_f[pl.ds(16 * i, 16)] = _zero16()
        return _
    lax.fori_loop(0, N // 16, zero_body, None)

    base = w * ROWS_PER_WORKER

    def chunk_body(c, _):
        pltpu.sync_copy(dst2d.at[pl.ds(base + 40 * c, 40)], dbuf)
        pltpu.sync_copy(ew2d.at[pl.ds(base + 40 * c, 40)], wbuf)

        def row_body(j, _):
            for g in range(8):
                d16 = dbuf[j, pl.ds(16 * g, 16)]
                w16 = wbuf[j, pl.ds(16 * g, 16)]
                plsc.addupdate_scatter(acc_d, [d16], w16)
            return _
        lax.fori_loop(0, 40, row_body, None)
        return _
    lax.fori_loop(0, ROWS_PER_WORKER // 40, chunk_body, None)

    @pl.when(w < 20)
    def _():
        pltpu.sync_copy(focus3d.at[w], fbuf)
        ones = jnp.ones((16,), jnp.float32)
        for g in range(8):
            f16 = fbuf[0, pl.ds(16 * g, 16)]
            plsc.addupdate_scatter(acc_f, [f16], ones)

    pltpu.sync_copy(acc_d, deg_out.at[w].at[0])
    pltpu.sync_copy(acc_f, cnt_out.at[w].at[0])


@functools.partial(
    pl.kernel,
    out_type=jax.ShapeDtypeStruct((2 * N, H // 2), jnp.float32),
    mesh=_mesh,
    compiler_params=pltpu.CompilerParams(needs_layout_passes=False, use_tc_tiling_on_sc=False),
    scratch_types=[
        pltpu.VMEM((2, 8, 128), jnp.int32),
        pltpu.VMEM((2, 8, 128), jnp.int32),
        pltpu.VMEM((2, 8, 128), jnp.float32),
        pltpu.VMEM((4, 128, H // 2), jnp.float32),
        pltpu.VMEM_SHARED((N, H // 2), jnp.float32),
        pltpu.SemaphoreType.DMA((4,)),
        pltpu.SemaphoreType.DMA((4,)),
        pltpu.SemaphoreType.DMA((2,)),
    ],
)
def _agg_kernel(y_cat, srcA, srcB, dst2d, ew2d, agg_out,
                sbuf, dbuf, wbuf, gbuf, acc_s, sem_g, sem_s, sem_i):
    ci = lax.axis_index("c")
    si = lax.axis_index("s")
    coff = ci * N

    nbase = pl.multiple_of(jnp.where(si == 15, N - NPT, NPT * si), 8)
    pltpu.sync_copy(y_cat.at[pl.ds(coff + nbase, NPT)],
                    acc_s.at[pl.ds(nbase, NPT)])
    plsc.subcore_barrier()

    ebase = si * ROWS_PER_TILE
    RPT = ROWS_PER_TILE

    def idx_issue(r0, q):
        @pl.when(ci == 0)
        def _():
            pltpu.async_copy(srcA.at[pl.ds(r0, 8)], sbuf.at[q], sem_i.at[q])

        @pl.when(ci == 1)
        def _():
            pltpu.async_copy(srcB.at[pl.ds(r0, 8)], sbuf.at[q], sem_i.at[q])
        pltpu.async_copy(dst2d.at[pl.ds(r0, 8)], dbuf.at[q], sem_i.at[q])
        pltpu.async_copy(ew2d.at[pl.ds(r0, 8)], wbuf.at[q], sem_i.at[q])

    def idx_wait(q):
        pltpu.make_async_copy(srcA.at[pl.ds(0, 8)], sbuf.at[q],
                              sem_i.at[q]).wait()
        pltpu.make_async_copy(dst2d.at[pl.ds(0, 8)], dbuf.at[q],
                              sem_i.at[q]).wait()
        pltpu.make_async_copy(ew2d.at[pl.ds(0, 8)], wbuf.at[q],
                              sem_i.at[q]).wait()

    def g_issue(q, jr, slot):
        pltpu.async_copy(y_cat.at[sbuf.at[q].at[jr]], gbuf.at[slot],
                         sem_g.at[slot])

    def g_wait(slot):
        pltpu.make_async_copy(y_cat.at[sbuf.at[0].at[0]], gbuf.at[slot],
                              sem_g.at[slot]).wait()

    def s_issue(q, jr, slot):
        pltpu.async_copy(gbuf.at[slot], acc_s.at[dbuf.at[q].at[jr]],
                         sem_s.at[slot], add=True)

    def s_drain(slot):
        pltpu.make_async_copy(gbuf.at[slot], acc_s.at[dbuf.at[0].at[0]],
                              sem_s.at[slot]).wait()

    def scale(q, jr, slot):
        def scale_body(i, _):
            for u4 in range(8):
                e = 8 * i + u4
                w16 = plsc.load_gather(
                    wbuf.at[q], [jnp.full((16,), jr, jnp.int32),
                                 jnp.full((16,), e, jnp.int32)])
                gbuf[slot, e, pl.ds(0, 16)] = \
                    gbuf[slot, e, pl.ds(0, 16)] * w16
                gbuf[slot, e, pl.ds(16, 16)] = \
                    gbuf[slot, e, pl.ds(16, 16)] * w16
            return _
        lax.fori_loop(0, 16, scale_body, None)

    idx_issue(ebase, 0)
    idx_wait(0)
    g_issue(0, 0, 0)
    g_issue(0, 1, 1)

    def block_body(it, _):
        for u in range(16):
            m = 16 * it + u
            slot = u % 4
            nslot = (u + 2) % 4
            q = (u // 8) % 2
            qn = ((u + 2) // 8) % 2
            jr = u % 8

            @pl.when(m >= 2)
            def _():
                s_drain(nslot)
            if u in (5, 13):
                @pl.when(m + 3 < RPT)
                def _(m=m):
                    idx_issue(pl.multiple_of(ebase + m + 3, 8),
                              (u // 8 + 1) % 2)
            if u in (6, 14):
                @pl.when(m + 2 < RPT)
                def _():
                    idx_wait((u // 8 + 1) % 2)

            @pl.when(m + 2 < RPT)
            def _():
                g_issue(qn, (u + 2) % 8, nslot)
            g_wait(slot)
            scale(q, jr, slot)
            s_issue(q, jr, slot)
        return _
    lax.fori_loop(0, RPT // 16, block_body, None)
    for slot in (2, 3):
        s_drain(slot)

    plsc.subcore_barrier()
    pltpu.sync_copy(acc_s.at[pl.ds(nbase, NPT)],
                    agg_out.at[pl.ds(coff + nbase, NPT)])


@functools.partial(
    pl.kernel,
    out_type=(jax.ShapeDtypeStruct((26, GPAD, H), jnp.float32),
              jax.ShapeDtypeStruct((32, 1, GPAD), jnp.float32)),
    mesh=_mesh,
    compiler_params=pltpu.CompilerParams(needs_layout_passes=False, use_tc_tiling_on_sc=False),
    scratch_types=[
        pltpu.VMEM((128, H), jnp.float32),
        pltpu.VMEM((1, 128), jnp.int32),
        pltpu.VMEM((GPAD,), jnp.float32),
        pltpu.VMEM_SHARED((13, GPAD, H), jnp.float32),
    ],
)
def _readout_kernel(*args):
    xs = args[:13]
    batch3d, z528 = args[13], args[14]
    sums_out, cnt_out = args[15], args[16]
    xbuf, ibuf, cntv, acc_r = args[17:]

    ci = lax.axis_index("c")
    si = lax.axis_index("s")
    w = ci * 16 + si

    @pl.when(si == 0)
    def _():
        for i in range(13):
            pltpu.sync_copy(z528, acc_r.at[i])

    z16i = jnp.zeros((16,), jnp.int32)

    def zero_body(i, _):
        cntv[pl.ds(16 * i, 16)] = _zero16()
        return _
    lax.fori_loop(0, GPAD // 16, zero_body, None)
    plsc.subcore_barrier()

    ones = jnp.ones((16,), jnp.float32)
    for b in range(WROWS):
        row = w * WROWS + b
        nb = pl.multiple_of(row * 128, 8)

        @pl.when(nb < N)
        def _(row=row, nb=nb):
            pltpu.sync_copy(batch3d.at[row], ibuf)

            @pl.when(nb + 128 <= N)
            def _():
                for i in range(13):
                    pltpu.sync_copy(xs[i].at[pl.ds(nb, 128)], xbuf)
                    pltpu.sync_copy(xbuf, acc_r.at[i].at[ibuf.at[0]], add=True)

            @pl.when(nb + 128 > N)
            def _():
                for i in range(13):
                    pltpu.sync_copy(xs[i].at[pl.ds(nb, N % 128)],
                                    xbuf.at[pl.ds(0, N % 128)])
                    pltpu.sync_copy(xbuf, acc_r.at[i].at[ibuf.at[0]], add=True)

            for g in range(8):
                b16 = ibuf[0, pl.ds(16 * g, 16)]
                plsc.addupdate_scatter(cntv, [b16], ones)

    plsc.subcore_barrier()
    pltpu.sync_copy(cntv, cnt_out.at[w].at[0])

    @pl.when(si < 13)
    def _():
        pltpu.sync_copy(acc_r.at[si], sums_out.at[ci * 13 + si])


def _reduce32_body(dp_ref, cp_ref, do_ref, co_ref):
    @pl.when(pl.program_id(1) == 0)
    def _():
        do_ref[...] = jnp.zeros_like(do_ref)
        co_ref[...] = jnp.zeros_like(co_ref)
    do_ref[...] += dp_ref[0]
    co_ref[...] += cp_ref[0]


def _first_body(nf_ref, cat_ref, degp_ref, cntp_ref, emb_ref, wf_ref, bf_ref,
                w1_ref, x0_ref, dis_ref, y1_ref):
    deg = degp_ref[...] + 1.0
    dis = lax.rsqrt(deg)
    bit = jnp.minimum(cntp_ref[...], 1.0)
    cat = cat_ref[...]
    iota = lax.broadcasted_iota(jnp.int32, (1, VOCAB), 1)
    onehot = (cat == iota).astype(jnp.float32)
    tbl = jnp.dot(emb_ref[...], wf_ref[pl.ds(F_IN, EMB), :],
                  preferred_element_type=jnp.float32)
    x = jnp.dot(nf_ref[...], wf_ref[pl.ds(0, F_IN), :],
                preferred_element_type=jnp.float32)
    x = x + jnp.dot(onehot, tbl, preferred_element_type=jnp.float32)
    x = x + bit * wf_ref[pl.ds(F_IN + EMB, 1), :] + bf_ref[...]
    x0 = jnp.maximum(x, 0.0)
    y = jnp.dot(x0, w1_ref[...], preferred_element_type=jnp.float32) * dis
    x0_ref[...] = x0
    dis_ref[...] = dis
    y1_ref[0] = y[:, :H // 2]
    y1_ref[1] = y[:, H // 2:]


def _layer_body(agg_ref, dis_ref, b_ref, wn_ref, x_ref, yn_ref):
    dis = dis_ref[...]
    a = jnp.concatenate([agg_ref[0], agg_ref[1]], axis=-1)
    x = jnp.maximum(a * dis + b_ref[...], 0.0)
    x_ref[...] = x
    yn = jnp.dot(x, wn_ref[...], preferred_element_type=jnp.float32) * dis
    yn_ref[0] = yn[:, :H // 2]
    yn_ref[1] = yn[:, H // 2:]


def _last_body(agg_ref, dis_ref, b_ref, x_ref):
    a = jnp.concatenate([agg_ref[0], agg_ref[1]], axis=-1)
    x_ref[...] = jnp.maximum(a * dis_ref[...] + b_ref[...], 0.0)


def _concat_body(*refs):
    out_ref = refs[-1]
    for i in range(13):
        out_ref[:, H * i:H * (i + 1)] = refs[i][...]


def _div_body(sums_ref, cnt_ref, out_ref):
    counts = jnp.sum(cnt_ref[...], axis=0)[:G]
    cmax = jnp.maximum(counts, 1.0)[:, None]
    for i in range(13):
        si = sums_ref[i, pl.ds(0, G), :] + sums_ref[13 + i, pl.ds(0, G), :]
        out_ref[:, pl.ds(H * i, H)] = si / cmax


_BN = 1000
_NG = N // _BN


def _full(shape):
    return pl.BlockSpec(shape, lambda i: tuple(0 for _ in shape))


def kernel(partial_graph_node_categorical_features, node_features, edge_index,
           edge_features, graph_to_focus_node_map, candidate_attachment_points,
           batch_index, embed_table, W_first, b_first, Ws, bs):
    f32 = jnp.float32
    src = edge_index[0].astype(jnp.int32)
    dst = edge_index[1].astype(jnp.int32)
    ew = edge_features.astype(f32)
    pad = E_PAD - E
    src2d = jnp.pad(src, (0, pad)).reshape(ER, 128)
    dst2d = jnp.pad(dst, (0, pad)).reshape(ER, 128)
    ew2d = jnp.pad(ew, (0, pad)).reshape(ER, 128)
    focus = jnp.concatenate([graph_to_focus_node_map,
                             candidate_attachment_points]).astype(jnp.int32)
    focus3d = focus.reshape(20, 1, 128)
    batch3d = jnp.pad(batch_index.astype(jnp.int32), (0, BR * 128 - N),
                      constant_values=G).reshape(BR, 1, 128)
    src2db = src2d + N
    z528 = jnp.zeros((GPAD, H), f32)

    deg_p, cnt_p = _stats_kernel(dst2d, ew2d, focus3d)
    deg_p = deg_p.reshape(32, N, 1)
    cnt_p = cnt_p.reshape(32, N, 1)
    deg_r, cnt_r = pl.pallas_call(
        _reduce32_body,
        grid=(_NG, 32),
        in_specs=[
            pl.BlockSpec((1, _BN, 1), lambda i, k: (k, i, 0)),
            pl.BlockSpec((1, _BN, 1), lambda i, k: (k, i, 0)),
        ],
        out_specs=[
            pl.BlockSpec((_BN, 1), lambda i, k: (i, 0)),
            pl.BlockSpec((_BN, 1), lambda i, k: (i, 0)),
        ],
        out_shape=[
            jax.ShapeDtypeStruct((N, 1), f32),
            jax.ShapeDtypeStruct((N, 1), f32),
        ],
    )(deg_p, cnt_p)

    cat2 = partial_graph_node_categorical_features.astype(jnp.int32)[:, None]
    nf = node_features.astype(f32)

    x0, dis, y1 = pl.pallas_call(
        _first_body,
        grid=(_NG,),
        in_specs=[
            pl.BlockSpec((_BN, F_IN), lambda i: (i, 0)),
            pl.BlockSpec((_BN, 1), lambda i: (i, 0)),
            pl.BlockSpec((_BN, 1), lambda i: (i, 0)),
            pl.BlockSpec((_BN, 1), lambda i: (i, 0)),
            _full((VOCAB, EMB)),
            _full((F_IN + EMB + 1, H)),
            _full((1, H)),
            _full((H, H)),
        ],
        out_specs=[
            pl.BlockSpec((_BN, H), lambda i: (i, 0)),
            pl.BlockSpec((_BN, 1), lambda i: (i, 0)),
            pl.BlockSpec((2, _BN, H // 2), lambda i: (0, i, 0)),
        ],
        out_shape=[
            jax.ShapeDtypeStruct((N, H), f32),
            jax.ShapeDtypeStruct((N, 1), f32),
            jax.ShapeDtypeStruct((2, N, H // 2), f32),
        ],
    )(nf, cat2, deg_r, cnt_r, embed_table.astype(f32),
      W_first.astype(f32), b_first.astype(f32)[None, :], Ws[0].astype(f32))

    xs = [x0]
    y = y1
    for i in range(1, L + 1):
        agg = _agg_kernel(y.reshape(2 * N, H // 2), src2d, src2db, dst2d, ew2d)
        agg = agg.reshape(2, N, H // 2)
        bias = bs[i - 1].astype(f32)[None, :]
        if i < L:
            xi, y = pl.pallas_call(
                _layer_body,
                grid=(_NG,),
                in_specs=[
                    pl.BlockSpec((2, _BN, H // 2), lambda i: (0, i, 0)),
                    pl.BlockSpec((_BN, 1), lambda i: (i, 0)),
                    _full((1, H)),
                    _full((H, H)),
                ],
                out_specs=[
                    pl.BlockSpec((_BN, H), lambda i: (i, 0)),
                    pl.BlockSpec((2, _BN, H // 2), lambda i: (0, i, 0)),
                ],
                out_shape=[
                    jax.ShapeDtypeStruct((N, H), f32),
                    jax.ShapeDtypeStruct((2, N, H // 2), f32),
                ],
            )(agg, dis, bias, Ws[i].astype(f32))
        else:
            xi = pl.pallas_call(
                _last_body,
                grid=(_NG,),
                in_specs=[
                    pl.BlockSpec((2, _BN, H // 2), lambda i: (0, i, 0)),
                    pl.BlockSpec((_BN, 1), lambda i: (i, 0)),
                    _full((1, H)),
                ],
                out_specs=pl.BlockSpec((_BN, H), lambda i: (i, 0)),
                out_shape=jax.ShapeDtypeStruct((N, H), f32),
            )(agg, dis, bias)
        xs.append(xi)

    sums_p, cnt_parts = _readout_kernel(*xs, batch3d, z528)
    cnt_parts = cnt_parts.reshape(32, GPAD)

    graph_representations = pl.pallas_call(
        _div_body,
        grid=(1,),
        in_specs=[_full((26, GPAD, H)), _full((32, GPAD))],
        out_specs=_full((G, 13 * H)),
        out_shape=jax.ShapeDtypeStruct((G, 13 * H), f32),
    )(sums_p, cnt_parts)

    node_representations = pl.pallas_call(
        _concat_body,
        grid=(_NG,),
        in_specs=[pl.BlockSpec((_BN, H), lambda i: (i, 0))] * 13,
        out_specs=pl.BlockSpec((_BN, 13 * H), lambda i: (i, 0)),
        out_shape=jax.ShapeDtypeStruct((N, 13 * H), f32),
    )(*xs)
    return (graph_representations, node_representations)
